# baseline (speedup 1.0000x reference)
import functools

import jax
import jax.numpy as jnp
from jax import lax
from jax.experimental import pallas as pl
from jax.experimental.pallas import tpu as pltpu
from jax.experimental.pallas import tpu_sc as plsc

NC = 2
NS = 16
NW = NC * NS
LANES = 16
NBUF = 4


def _build(B, L, D):
    assert D % LANES == 0 and B % NW == 0
    H1 = min(128, L)
    H2 = L - H1
    assert 0 < H2 <= 128
    S = B // NW
    DJ = D // LANES

    mesh = plsc.VectorSubcoreMesh(core_axis_name="c", subcore_axis_name="s")

    @functools.partial(
        pl.kernel,
        out_type=jax.ShapeDtypeStruct((B, L, D), jnp.float32),
        mesh=mesh,
        scratch_types=(
            [pltpu.VMEM((L,), jnp.int32) for _ in range(NBUF)]
            + [pltpu.VMEM((L, D), jnp.float32) for _ in range(NBUF)]
            + [pltpu.VMEM((L, D), jnp.float32)]
            + [pltpu.SemaphoreType.DMA for _ in range(2 * NBUF)]
        ),
        compiler_params=pltpu.CompilerParams(use_tc_tiling_on_sc=False),
    )
    def emb(x_hbm, table_hbm, pe_hbm, out_hbm, *refs):
        idx = refs[0:NBUF]
        rows = refs[NBUF:2 * NBUF]
        pe_v = refs[2 * NBUF]
        gsem = refs[2 * NBUF + 1:2 * NBUF + 1 + NBUF]
        osem = refs[2 * NBUF + 1 + NBUF:]

        wid = lax.axis_index("s") * NC + lax.axis_index("c")
        seq0 = wid * S

        pltpu.sync_copy(pe_hbm, pe_v)

        def start_gather(b, s):
            g = seq0 + s
            pltpu.sync_copy(x_hbm.at[pl.ds(g * L, L)], idx[b])
            pltpu.async_copy(table_hbm.at[idx[b].at[pl.ds(0, H1)]],
                             rows[b].at[pl.ds(0, H1)], gsem[b])
            pltpu.async_copy(table_hbm.at[idx[b].at[pl.ds(H1, H2)]],
                             rows[b].at[pl.ds(H1, H2)], gsem[b])

        def wait_gather(b):
            pltpu.make_async_copy(table_hbm.at[idx[b].at[pl.ds(0, H1)]],
                                  rows[b].at[pl.ds(0, H1)], gsem[b]).wait()
            pltpu.make_async_copy(table_hbm.at[idx[b].at[pl.ds(H1, H2)]],
                                  rows[b].at[pl.ds(H1, H2)], gsem[b]).wait()

        def start_out(b, s):
            pltpu.async_copy(rows[b], out_hbm.at[seq0 + s], osem[b])

        def wait_out(b, s):
            pltpu.make_async_copy(rows[b], out_hbm.at[seq0 + s],
                                  osem[b]).wait()

        for b in range(NBUF):
            start_gather(b, b)

        @pl.loop(0, S // NBUF)
        def _(it):
            s_base = it * NBUF
            for b in range(NBUF):
                s = s_base + b
                wait_gather(b)

                @pl.loop(0, L)
                def _(i):
                    for j in range(DJ):
                        sl = pl.ds(j * LANES, LANES)
                        rows[b][i, sl] = rows[b][i, sl] + pe_v[i, sl]

                start_out(b, s)

                @pl.when(s + NBUF < S)
                def _():
                    wait_out(b, s)
                    start_gather(b, s + NBUF)

        for b in range(NBUF):
            wait_out(b, S - NBUF + b)

    return emb


def kernel(x, table, pe):
    B, L = x.shape
    V, D = table.shape
    emb = _build(B, L, D)
    tab2 = jnp.pad(table, ((0, 0), (0, D))).reshape(2 * V, D)
    x2 = (x.astype(jnp.int32) * 2).reshape(B * L)
    return emb(x2, tab2, pe.astype(jnp.float32))


# speedup vs baseline: 1.1200x; 1.1200x over previous
import functools

import jax
import jax.numpy as jnp
from jax import lax
from jax.experimental import pallas as pl
from jax.experimental.pallas import tpu as pltpu
from jax.experimental.pallas import tpu_sc as plsc

NC = 2
NS = 16
NW = NC * NS
LANES = 16
NBUF = 4


def _build(B, L, D):
    assert D % LANES == 0 and B % NW == 0
    H1 = min(128, L)
    H2 = L - H1
    assert 0 < H2 <= 128
    S = B // NW
    DJ = D // LANES

    mesh = plsc.VectorSubcoreMesh(core_axis_name="c", subcore_axis_name="s")

    @functools.partial(
        pl.kernel,
        out_type=jax.ShapeDtypeStruct((B, L, 2 * D), jnp.float32),
        mesh=mesh,
        scratch_types=(
            [pltpu.VMEM((L,), jnp.int32) for _ in range(NBUF)]
            + [pltpu.VMEM((L, D), jnp.float32) for _ in range(NBUF)]
            + [pltpu.VMEM((L, D), jnp.float32)]
            + [pltpu.SemaphoreType.DMA for _ in range(2 * NBUF)]
        ),
        compiler_params=pltpu.CompilerParams(use_tc_tiling_on_sc=False),
    )
    def emb(x_hbm, table_hbm, pe_hbm, out_hbm, *refs):
        idx = refs[0:NBUF]
        rows = refs[NBUF:2 * NBUF]
        pe_v = refs[2 * NBUF]
        gsem = refs[2 * NBUF + 1:2 * NBUF + 1 + NBUF]
        osem = refs[2 * NBUF + 1 + NBUF:]

        wid = lax.axis_index("s") * NC + lax.axis_index("c")
        seq0 = wid * S

        pltpu.sync_copy(pe_hbm, pe_v)

        def start_gather(b, s):
            g = seq0 + s
            pltpu.sync_copy(x_hbm.at[pl.ds(g * L, L)], idx[b])
            pltpu.async_copy(table_hbm.at[idx[b].at[pl.ds(0, H1)]],
                             rows[b].at[pl.ds(0, H1)], gsem[b])
            pltpu.async_copy(table_hbm.at[idx[b].at[pl.ds(H1, H2)]],
                             rows[b].at[pl.ds(H1, H2)], gsem[b])

        def wait_gather(b):
            pltpu.make_async_copy(table_hbm.at[idx[b].at[pl.ds(0, H1)]],
                                  rows[b].at[pl.ds(0, H1)], gsem[b]).wait()
            pltpu.make_async_copy(table_hbm.at[idx[b].at[pl.ds(H1, H2)]],
                                  rows[b].at[pl.ds(H1, H2)], gsem[b]).wait()

        def start_out(b, s):
            pltpu.async_copy(rows[b], out_hbm.at[seq0 + s].at[:, pl.ds(0, D)],
                             osem[b])

        def wait_out(b, s):
            pltpu.make_async_copy(rows[b],
                                  out_hbm.at[seq0 + s].at[:, pl.ds(0, D)],
                                  osem[b]).wait()

        for b in range(NBUF):
            start_gather(b, b)

        @pl.loop(0, S // NBUF)
        def _(it):
            s_base = it * NBUF
            for b in range(NBUF):
                s = s_base + b
                wait_gather(b)

                @pl.loop(0, L)
                def _(i):
                    for j in range(DJ):
                        sl = pl.ds(j * LANES, LANES)
                        rows[b][i, sl] = rows[b][i, sl] + pe_v[i, sl]

                start_out(b, s)

                @pl.when(s + NBUF < S)
                def _():
                    wait_out(b, s)
                    start_gather(b, s + NBUF)

        for b in range(NBUF):
            wait_out(b, S - NBUF + b)

    return emb


def kernel(x, table, pe):
    B, L = x.shape
    V, D = table.shape
    emb = _build(B, L, D)
    tab2 = jnp.pad(table, ((0, 0), (0, D))).reshape(2 * V, D)
    x2 = (x.astype(jnp.int32) * 2).reshape(B * L)
    return emb(x2, tab2, pe.astype(jnp.float32))[:, :, :D]
